# Initial kernel scaffold; baseline (speedup 1.0000x reference)
import jax
import jax.numpy as jnp
from jax.experimental import pallas as pl


def kernel(x_user, x_item, params, src_u2i, dst_u2i, src_i2u, dst_i2u, label_src, label_dst):
    raise NotImplementedError("write your pallas kernel here")


# speedup vs baseline: 7.5133x; 7.5133x over previous
import functools

import jax
import jax.numpy as jnp
from jax import lax
from jax.experimental import pallas as pl
from jax.experimental.pallas import tpu as pltpu
from jax.experimental.pallas import tpu_sc as plsc

F32 = jnp.float32
I32 = jnp.int32
EPS = 1e-16
NEG_SLOPE = 0.2

NC = 2
NS = 16
NW = NC * NS
CHUNK = 128


def _rup(x, m):
    return (x + m - 1) // m * m


def _tc_transform(x, w, b, n_out_rows, split_hi):
    n_rows = x.shape[0]
    blk = 1024
    grid = (n_out_rows // blk,)
    assert pl.cdiv(n_rows, blk) == grid[0]

    def body(x_ref, w_ref, b_ref, o1_ref, o2_ref, *rest):
        h = jnp.dot(x_ref[...], w_ref[...], preferred_element_type=F32)
        h = h + b_ref[...]
        o1_ref[...] = h[:, :32]
        o2_ref[...] = h[:, 32:]
        if split_hi:
            rest[0][...] = h[:, 16:32]

    outs = [jax.ShapeDtypeStruct((n_out_rows, 32), F32),
            jax.ShapeDtypeStruct((n_out_rows, 32), F32)]
    out_specs = [pl.BlockSpec((blk, 32), lambda i: (i, 0)),
                 pl.BlockSpec((blk, 32), lambda i: (i, 0))]
    if split_hi:
        outs.append(jax.ShapeDtypeStruct((n_out_rows, 16), F32))
        out_specs.append(pl.BlockSpec((blk, 16), lambda i: (i, 0)))
    return pl.pallas_call(
        body,
        grid=grid,
        in_specs=[pl.BlockSpec((blk, x.shape[1]), lambda i: (i, 0)),
                  pl.BlockSpec(x.shape[1:2] + (64,), lambda i: (0, 0)),
                  pl.BlockSpec((1, 64), lambda i: (0, 0))],
        out_specs=out_specs,
        out_shape=outs,
    )(x, w, b.reshape(1, 64))


def _tc_finalize(num_parts, den_parts, bias, relu, w, b, out_widths,
                 num_parts_hi=None):
    np_, den = num_parts, den_parts
    n_rows = den.shape[1]
    blk = 1024
    grid = (n_rows // blk,)
    wdim = w.shape[1]
    has_hi = num_parts_hi is not None

    def body(*refs):
        if has_hi:
            nlo_ref, nhi_ref, den_ref, b1_ref, w_ref, b2_ref = refs[:6]
            out_refs = refs[6:]
            num = jnp.concatenate(
                [nlo_ref[0] + nlo_ref[1], nhi_ref[0] + nhi_ref[1]], axis=-1)
        else:
            n_ref, den_ref, b1_ref, w_ref, b2_ref = refs[:5]
            out_refs = refs[5:]
            num = n_ref[0] + n_ref[1]
        d = den_ref[0] + den_ref[1]
        z = num / (d[:, None] + EPS) + b1_ref[...]
        if relu:
            z = jnp.maximum(z, 0.0)
        h = jnp.dot(z, w_ref[...], preferred_element_type=F32) + b2_ref[...]
        off = 0
        for o_ref, wd in zip(out_refs, out_widths):
            o_ref[...] = h[:, off:off + wd]
            off += wd

    in_arrays = []
    in_specs = []
    if has_hi:
        in_arrays += [np_, num_parts_hi]
        in_specs += [pl.BlockSpec((2, blk, 16), lambda i: (0, i, 0))] * 2
    else:
        in_arrays.append(np_)
        in_specs.append(pl.BlockSpec((2, blk, 32), lambda i: (0, i, 0)))
    in_arrays += [den, bias.reshape(1, 32), w, b.reshape(1, wdim)]
    in_specs += [pl.BlockSpec((2, blk), lambda i: (0, i)),
                 pl.BlockSpec((1, 32), lambda i: (0, 0)),
                 pl.BlockSpec((32, wdim), lambda i: (0, 0)),
                 pl.BlockSpec((1, wdim), lambda i: (0, 0))]
    outs = [jax.ShapeDtypeStruct((n_rows, wd), F32) for wd in out_widths]
    out_specs = [pl.BlockSpec((blk, wd), lambda i: (i, 0)) for wd in out_widths]
    return pl.pallas_call(
        body, grid=grid, in_specs=in_specs, out_specs=out_specs,
        out_shape=outs,
    )(*in_arrays)


def _iota16():
    return lax.iota(I32, 16)


def _make_conv_sc(mode, ep, ndp, nsl, nsr):
    w = 32 if mode == 'full' else 16
    nchunk = ep // (NW * CHUNK)
    assert nchunk * NW * CHUNK == ep
    zstride = ndp // NS
    nzc = zstride // CHUNK
    assert nzc * CHUNK * NS == ndp
    mesh = plsc.VectorSubcoreMesh(core_axis_name="c", subcore_axis_name="s",
                                  num_cores=NC, num_subcores=NS)

    out_type = [jax.ShapeDtypeStruct((NC, ndp, w), F32),
                jax.ShapeDtypeStruct((NC, ndp), F32)]
    if mode == 'lo':
        out_type.append(jax.ShapeDtypeStruct((ep,), F32))

    scratch = dict(
        sidx=pltpu.VMEM((1, CHUNK), I32),
        didx=pltpu.VMEM((1, CHUNK), I32),
        xlr=pltpu.VMEM((CHUNK, 32 if mode != 'hi' else 16), F32),
        orow=pltpu.VMEM((CHUNK, w), F32),
        exr=pltpu.VMEM((1, CHUNK), F32),
        attv=pltpu.VMEM((1, 32), F32),
        sem1=pltpu.SemaphoreType.DMA,
        sem2=pltpu.SemaphoreType.DMA,
        numS=pltpu.VMEM_SHARED((ndp, w), F32),
        denS=pltpu.VMEM_SHARED((ndp,), F32),
    )
    if mode != 'hi':
        scratch['xrr'] = pltpu.VMEM((CHUNK, 32), F32)

    def body(*refs):
        if mode == 'hi':
            (xl_h, ex_h, src_h, dst_h, zr_h, zv_h,
             num_h, den_h) = refs[:8]
            sc = dict(zip(
                ['sidx', 'didx', 'xlr', 'orow', 'exr', 'attv', 'sem1',
                 'sem2', 'numS', 'denS'], refs[8:]))
        else:
            (xl_h, xr_h, att_h, src_h, dst_h, zr_h, zv_h), rest = refs[:7], refs[7:]
            if mode == 'lo':
                num_h, den_h, ex_h = rest[:3]
                rest = rest[3:]
            else:
                num_h, den_h = rest[:2]
                rest = rest[2:]
            sc = dict(zip(
                ['sidx', 'didx', 'xlr', 'orow', 'exr', 'attv', 'sem1',
                 'sem2', 'numS', 'denS', 'xrr'], rest))
        c = lax.axis_index("c")
        s = lax.axis_index("s")
        wid = c * NS + s
        sidx, didx, xlr, orow = sc['sidx'], sc['didx'], sc['xlr'], sc['orow']
        exr, attv, numS, denS = sc['exr'], sc['attv'], sc['numS'], sc['denS']
        sem1, sem2 = sc['sem1'], sc['sem2']

        pltpu.sync_copy(zr_h, orow)
        pltpu.sync_copy(zv_h, exr.at[0])
        @pl.loop(0, nzc)
        def _zero(r):
            base = s * zstride + r * CHUNK
            pltpu.sync_copy(orow, numS.at[pl.ds(base, CHUNK)])
            pltpu.sync_copy(exr.at[0], denS.at[pl.ds(base, CHUNK)])
        if mode != 'hi':
            pltpu.sync_copy(att_h, attv.at[0, pl.ds(0, 32)])
            att_lo = attv[0, pl.ds(0, 16)]
            att_hi = attv[0, pl.ds(16, 16)]
        plsc.subcore_barrier()

        @pl.loop(0, nchunk)
        def _chunk(j):
            base = (wid * nchunk + j) * CHUNK
            pltpu.sync_copy(src_h.at[pl.ds(base, CHUNK)], sidx.at[0])
            pltpu.sync_copy(dst_h.at[pl.ds(base, CHUNK)], didx.at[0])
            d1 = pltpu.async_copy(xl_h.at[sidx.at[0]], xlr, sem1)
            if mode != 'hi':
                d2 = pltpu.async_copy(xr_h.at[didx.at[0]], sc['xrr'], sem2)
            else:
                pltpu.sync_copy(ex_h.at[pl.ds(base, CHUNK)], exr.at[0])
            d1.wait()
            if mode != 'hi':
                d2.wait()
            for g in range(CHUNK // 16):
                ei = _iota16() + g * 16
                if mode != 'hi':
                    score = jnp.zeros((16,), F32)
                    for k in range(32):
                        kv = jnp.full((16,), k, I32)
                        a = plsc.load_gather(xlr, [ei, kv])
                        bb = plsc.load_gather(sc['xrr'], [ei, kv])
                        v = a + bb
                        v = jnp.maximum(v, NEG_SLOPE * v)
                        coef = att_lo[k] if k < 16 else att_hi[k - 16]
                        score = score + coef * v
                    ex = jnp.exp(score)
                    exr[0, pl.ds(g * 16, 16)] = ex
                else:
                    ex = exr[0, pl.ds(g * 16, 16)]
                for k in range(w):
                    kv = jnp.full((16,), k, I32)
                    a = plsc.load_gather(xlr, [ei, kv])
                    plsc.store_scatter(orow, [ei, kv], ex * a)
            pltpu.sync_copy(orow, numS.at[didx.at[0]], add=True)
            if mode != 'hi':
                pltpu.sync_copy(exr.at[0], denS.at[didx.at[0]], add=True)
            if mode == 'lo':
                pltpu.sync_copy(exr.at[0], ex_h.at[pl.ds(base, CHUNK)])

        plsc.subcore_barrier()
        pltpu.sync_copy(numS.at[pl.ds(s * zstride, zstride)],
                        num_h.at[c, pl.ds(s * zstride, zstride)])
        pltpu.sync_copy(denS.at[pl.ds(s * zstride, zstride)],
                        den_h.at[c, pl.ds(s * zstride, zstride)])

    scratch_list = [scratch[k] for k in
                    (['sidx', 'didx', 'xlr', 'orow', 'exr', 'attv', 'sem1',
                      'sem2', 'numS', 'denS'] +
                     (['xrr'] if mode != 'hi' else []))]
    return pl.kernel(body, out_type=tuple(out_type), mesh=mesh,
                     scratch_types=scratch_list,
                     compiler_params=pltpu.CompilerParams(
                         needs_layout_passes=False,
                         use_tc_tiling_on_sc=False))


def _make_decoder_sc(lp, up, ip):
    nchunk = lp // (NW * CHUNK)
    assert nchunk * NW * CHUNK == lp
    mesh = plsc.VectorSubcoreMesh(core_axis_name="c", subcore_axis_name="s",
                                  num_cores=NC, num_subcores=NS)

    def body(a_h, b_h, w2_h, ls_h, ld_h, out_h,
             lsx, ldx, ar, br, pr, w2v, sem1, sem2):
        c = lax.axis_index("c")
        s = lax.axis_index("s")
        wid = c * NS + s
        pltpu.sync_copy(w2_h, w2v.at[0, pl.ds(0, 48)])
        w2_lo = w2v[0, pl.ds(0, 16)]
        w2_hi = w2v[0, pl.ds(16, 16)]
        w2_b2 = w2v[0, pl.ds(32, 16)]

        @pl.loop(0, nchunk)
        def _chunk(j):
            base = (wid * nchunk + j) * CHUNK
            pltpu.sync_copy(ls_h.at[pl.ds(base, CHUNK)], lsx.at[0])
            pltpu.sync_copy(ld_h.at[pl.ds(base, CHUNK)], ldx.at[0])
            d1 = pltpu.async_copy(a_h.at[lsx.at[0]], ar, sem1)
            d2 = pltpu.async_copy(b_h.at[ldx.at[0]], br, sem2)
            d1.wait()
            d2.wait()
            for g in range(CHUNK // 16):
                ei = _iota16() + g * 16
                acc = jnp.full((16,), w2_b2[0], F32)
                for k in range(32):
                    kv = jnp.full((16,), k, I32)
                    a = plsc.load_gather(ar, [ei, kv])
                    bb = plsc.load_gather(br, [ei, kv])
                    h = jnp.maximum(a + bb, 0.0)
                    coef = w2_lo[k] if k < 16 else w2_hi[k - 16]
                    acc = acc + coef * h
                pr[0, pl.ds(g * 16, 16)] = acc
            pltpu.sync_copy(pr.at[0], out_h.at[pl.ds(base, CHUNK)])

    return pl.kernel(
        body,
        out_type=jax.ShapeDtypeStruct((lp,), F32),
        mesh=mesh,
        scratch_types=[
            pltpu.VMEM((1, CHUNK), I32),
            pltpu.VMEM((1, CHUNK), I32),
            pltpu.VMEM((CHUNK, 32), F32),
            pltpu.VMEM((CHUNK, 32), F32),
            pltpu.VMEM((1, CHUNK), F32),
            pltpu.VMEM((1, 48), F32),
            pltpu.SemaphoreType.DMA,
            pltpu.SemaphoreType.DMA,
        ],
        compiler_params=pltpu.CompilerParams(
            needs_layout_passes=False, use_tc_tiling_on_sc=False))


def kernel(x_user, x_item, params, src_u2i, dst_u2i, src_i2u, dst_i2u,
           label_src, label_dst):
    n_user, d_in = x_user.shape
    n_item = x_item.shape[0]
    e = src_u2i.shape[0]
    l = label_src.shape[0]
    up = _rup(n_user + 1, 1024)
    ip = _rup(n_item + 1, 1024)
    ep = _rup(e, NW * CHUNK)
    lp = _rup(l, NW * CHUNK)

    p = params

    def pad_edges(src, dst, n_dst):
        pad = ep - e
        src = jnp.concatenate([src.astype(I32), jnp.zeros((pad,), I32)])
        dst = jnp.concatenate([dst.astype(I32),
                               jnp.full((pad,), n_dst, I32)])
        return src, dst

    s_u2i, d_u2i = pad_edges(src_u2i, dst_u2i, n_item)
    s_i2u, d_i2u = pad_edges(src_i2u, dst_i2u, n_user)
    lpad = lp - l
    ls = jnp.concatenate([label_src.astype(I32), jnp.zeros((lpad,), I32)])
    ld = jnp.concatenate([label_dst.astype(I32), jnp.zeros((lpad,), I32)])

    z_rows32 = jnp.zeros((CHUNK, 32), F32)
    z_rows16 = jnp.zeros((CHUNK, 16), F32)
    z_vec = jnp.zeros((CHUNK,), F32)

    w_u1 = jnp.concatenate([p['c1_i2u']['Wr'], p['c1_u2i']['Wl']], axis=1)
    b_u1 = jnp.concatenate([p['c1_i2u']['br'], p['c1_u2i']['bl']])
    xr1_i2u, xl1_u2i = _tc_transform(x_user, w_u1, b_u1, up, False)
    w_i1 = jnp.concatenate([p['c1_i2u']['Wl'], p['c1_u2i']['Wr']], axis=1)
    b_i1 = jnp.concatenate([p['c1_i2u']['bl'], p['c1_u2i']['br']])
    xl1_i2u, xr1_u2i, xl1_i2u_hi = _tc_transform(x_item, w_i1, b_i1, ip, True)

    conv_users_lo = _make_conv_sc('lo', ep, up, ip, up)
    conv_users_hi = _make_conv_sc('hi', ep, up, ip, up)
    conv_items = _make_conv_sc('full', ep, ip, up, ip)

    att1u = p['c1_i2u']['att']
    nlo_u1, den_u1, ex_u1 = conv_users_lo(
        xl1_i2u, xr1_i2u, att1u, s_i2u, d_i2u, z_rows16, z_vec)
    nhi_u1, _, = conv_users_hi(
        xl1_i2u_hi, ex_u1, s_i2u, d_i2u, z_rows16, z_vec)
    num_i1, den_i1 = conv_items(
        xl1_u2i, xr1_u2i, p['c1_u2i']['att'], s_u2i, d_u2i, z_rows32, z_vec)

    w_u2 = jnp.concatenate([p['c2_i2u']['Wr'], p['c2_u2i']['Wl']], axis=1)
    b_u2 = jnp.concatenate([p['c2_i2u']['br'], p['c2_u2i']['bl']])
    xr2_i2u, xl2_u2i = _tc_finalize(
        nlo_u1, den_u1, p['c1_i2u']['bias'], True, w_u2, b_u2, (32, 32),
        num_parts_hi=nhi_u1)
    w_i2 = jnp.concatenate([p['c2_i2u']['Wl'], p['c2_u2i']['Wr']], axis=1)
    b_i2 = jnp.concatenate([p['c2_i2u']['bl'], p['c2_u2i']['br']])
    xl2_i2u, xr2_u2i = _tc_finalize(
        num_i1, den_i1, p['c1_u2i']['bias'], True, w_i2, b_i2, (32, 32))
    w_i2_hi = p['c2_i2u']['Wl'][:, 16:32]
    b_i2_hi = p['c2_i2u']['bl'][16:32]
    xl2_i2u_hi, = _tc_finalize(
        num_i1, den_i1, p['c1_u2i']['bias'], True, w_i2_hi, b_i2_hi, (16,))

    att2u = p['c2_i2u']['att']
    nlo_u2, den_u2, ex_u2 = conv_users_lo(
        xl2_i2u, xr2_i2u, att2u, s_i2u, d_i2u, z_rows16, z_vec)
    nhi_u2, _, = conv_users_hi(
        xl2_i2u_hi, ex_u2, s_i2u, d_i2u, z_rows16, z_vec)
    num_i2, den_i2 = conv_items(
        xl2_u2i, xr2_u2i, p['c2_u2i']['att'], s_u2i, d_u2i, z_rows32, z_vec)

    a_tab, = _tc_finalize(
        nlo_u2, den_u2, p['c2_i2u']['bias'], False, p['dec_W1'][:32],
        jnp.zeros((32,), F32), (32,), num_parts_hi=nhi_u2)
    b_tab, = _tc_finalize(
        num_i2, den_i2, p['c2_u2i']['bias'], False, p['dec_W1'][32:],
        p['dec_b1'], (32,))

    w2b = jnp.zeros((48,), F32).at[:32].set(p['dec_W2'][:, 0]).at[32].set(
        p['dec_b2'][0])
    pred_pad = _make_decoder_sc(lp, up, ip)(a_tab, b_tab, w2b, ls, ld)
    pred = pred_pad[:l]
    mask = jnp.ones((l,), dtype=bool)
    return pred, mask
